# baseline (speedup 1.0000x reference)
import functools

import jax
import jax.numpy as jnp
from jax import lax
from jax.experimental import pallas as pl
from jax.experimental.pallas import tpu as pltpu
from jax.experimental.pallas import tpu_sc as plsc

N = 100000
E = 3200000
D_IN = 128
D = 16
DE = 4
NC = 2
NS = 16
NW = NC * NS
SUB = 128
KSUB = 2
C = SUB * KSUB
NCHUNK = E // C
ZR = 1000
B = 1000
NB = N // B
EPS = 1e-5
_SLOPE = 0.2


def _prologue_body(x_ref, wet_ref, be_ref, lw_ref, lb_ref, wlt_ref, bl_ref,
                   wrt_ref, br_ref, h0_ref, xl_ref, xr_ref):
    h = jnp.dot(x_ref[...], wet_ref[...], preferred_element_type=jnp.float32)
    h = h + be_ref[...]
    mu = jnp.mean(h, axis=1, keepdims=True)
    var = jnp.mean((h - mu) ** 2, axis=1, keepdims=True)
    h = (h - mu) / jnp.sqrt(var + EPS) * lw_ref[...] + lb_ref[...]
    h0 = jnp.maximum(h, 0.0)
    h0_ref[...] = h0
    xl_ref[...] = jnp.dot(h0, wlt_ref[...],
                          preferred_element_type=jnp.float32) + bl_ref[...]
    xr_ref[...] = jnp.dot(h0, wrt_ref[...],
                          preferred_element_type=jnp.float32) + br_ref[...]


def _prologue(x, W_embed, b_embed, ln0_w, ln0_b, W_l, b_l, W_r, b_r):
    full = lambda shape: pl.BlockSpec(shape, lambda i: (0, 0))
    out = jax.ShapeDtypeStruct((N, D), jnp.float32)
    return pl.pallas_call(
        _prologue_body,
        grid=(NB,),
        in_specs=[
            pl.BlockSpec((B, D_IN), lambda i: (i, 0)),
            full((D_IN, D)), full((1, D)), full((1, D)), full((1, D)),
            full((D, D)), full((1, D)), full((D, D)), full((1, D)),
        ],
        out_specs=[pl.BlockSpec((B, D), lambda i: (i, 0))] * 3,
        out_shape=[out, out, out],
    )(x, W_embed.T, b_embed.reshape(1, D), ln0_w.reshape(1, D),
      ln0_b.reshape(1, D), W_l.T, b_l.reshape(1, D), W_r.T, b_r.reshape(1, D))


ND = N // 16


def _sc_body(xl_hbm, xr_hbm, src_hbm, dst_hbm, dsthi_hbm, ea_hbm, we_hbm,
             att_hbm, zeros_hbm,
             agg_out, den_out,
             agg_sh, den_sh, srcv, dstv, dsthiv, eav, xlv, xrv,
             wev, attv, sem_l, sem_r):
    cid = lax.axis_index("c")
    sid = lax.axis_index("s")
    wid = sid * NC + cid

    pltpu.sync_copy(we_hbm, wev)
    pltpu.sync_copy(att_hbm, attv)
    att_vec = attv[...]
    we_col = [wev[pl.ds(16 * k, 16)] for k in range(DE)]

    iota = lax.iota(jnp.int32, 16)

    @pl.loop(sid, N // ZR, step=NS)
    def _init(i):
        pltpu.sync_copy(zeros_hbm.at[pl.ds(0, ZR)],
                        agg_sh.at[pl.ds(i * ZR, ZR)])

    @pl.when(sid == 0)
    def _init_den():
        pltpu.sync_copy(zeros_hbm, den_sh)

    plsc.subcore_barrier()

    @pl.loop(wid, NCHUNK, step=NW)
    def _chunk(ch):
        row0 = ch * KSUB
        pltpu.sync_copy(src_hbm.at[pl.ds(row0, KSUB)], srcv)
        pltpu.sync_copy(dst_hbm.at[pl.ds(row0, KSUB)], dstv)
        pltpu.sync_copy(dsthi_hbm.at[pl.ds(row0, KSUB)], dsthiv)
        pltpu.sync_copy(ea_hbm.at[pl.ds(ch * (C * DE // 128), C * DE // 128)],
                        eav)
        for j in range(KSUB):
            pltpu.async_copy(xl_hbm.at[srcv.at[j]],
                             xlv.at[pl.ds(j * SUB, SUB)], sem_l)
            pltpu.async_copy(xr_hbm.at[dstv.at[j]],
                             xrv.at[pl.ds(j * SUB, SUB)], sem_r)
        for j in range(KSUB):
            pltpu.make_async_copy(xl_hbm.at[srcv.at[j]],
                                  xlv.at[pl.ds(j * SUB, SUB)], sem_l).wait()
            pltpu.make_async_copy(xr_hbm.at[dstv.at[j]],
                                  xrv.at[pl.ds(j * SUB, SUB)], sem_r).wait()

        @pl.loop(0, C // 16)
        def _group(g):
            e0 = g * 16
            lanes = dstv[g >> 3, pl.ds((g & 7) * 16, 16)] & 15
            av = jnp.zeros((16,), jnp.float32)
            xls = []
            for sub4 in range(4):
                q = g * 4 + sub4
                ea_vec = eav[q >> 3, pl.ds((q & 7) * 16, 16)]
                for q in range(4):
                    r = sub4 * 4 + q
                    xl = xlv[e0 + r]
                    xr = xrv[e0 + r]
                    s = xl + xr
                    for k in range(DE):
                        s = s + ea_vec[q * DE + k] * we_col[k]
                    m = jnp.maximum(s, s * _SLOPE)
                    t = m * att_vec
                    lvs = [t[l] for l in range(16)]
                    while len(lvs) > 1:
                        lvs = [lvs[i] + lvs[i + 1]
                               for i in range(0, len(lvs), 2)]
                    alpha = lvs[0]
                    av = jnp.where(iota == r, alpha, av)
                    xls.append(xl)
            p = jnp.exp(av)
            for r in range(16):
                xlv[e0 + r] = xls[r] * p[r]
                xrv[e0 + r] = jnp.where(iota == lanes[r], p[r], 0.0)

        for j in range(KSUB):
            pltpu.sync_copy(xlv.at[pl.ds(j * SUB, SUB)],
                            agg_sh.at[dstv.at[j]], add=True)
            pltpu.sync_copy(xrv.at[pl.ds(j * SUB, SUB)],
                            den_sh.at[dsthiv.at[j]], add=True)

    plsc.subcore_barrier()

    @pl.loop(sid, N // ZR, step=NS)
    def _dump(i):
        pltpu.sync_copy(agg_sh.at[pl.ds(i * ZR, ZR)],
                        agg_out.at[cid, pl.ds(i * ZR, ZR)])

    @pl.when(sid == 0)
    def _dump_den():
        pltpu.sync_copy(den_sh, den_out.at[cid])


def _sc_edge_pass(x_l, x_r, src2d, dst2d, dsthi2d, edge_attr, W_e, att,
                  zeros):
    mesh = plsc.VectorSubcoreMesh(core_axis_name="c", subcore_axis_name="s")
    fn = pl.kernel(
        _sc_body,
        out_type=[jax.ShapeDtypeStruct((NC, N, D), jnp.float32),
                  jax.ShapeDtypeStruct((NC, ND, 16), jnp.float32)],
        mesh=mesh,
        compiler_params=pltpu.CompilerParams(use_tc_tiling_on_sc=False),
        scratch_types=[
            pltpu.VMEM_SHARED((N, D), jnp.float32),
            pltpu.VMEM_SHARED((ND, 16), jnp.float32),
            pltpu.VMEM((KSUB, SUB), jnp.int32),
            pltpu.VMEM((KSUB, SUB), jnp.int32),
            pltpu.VMEM((KSUB, SUB), jnp.int32),
            pltpu.VMEM((C * DE // 128, 128), jnp.float32),
            pltpu.VMEM((C, D), jnp.float32),
            pltpu.VMEM((C, D), jnp.float32),
            pltpu.VMEM((D * DE,), jnp.float32),
            pltpu.VMEM((D,), jnp.float32),
            pltpu.SemaphoreType.DMA,
            pltpu.SemaphoreType.DMA,
        ],
    )
    return fn(x_l, x_r, src2d, dst2d, dsthi2d, edge_attr, W_e, att, zeros)


def _epi_a_body(agg_ref, den_ref, xl_ref, xr_ref, att_ref, gb_ref,
                a_ref, acc_ref):
    i = pl.program_id(0)
    xl = xl_ref[...]
    s = xl + xr_ref[...]
    m = jnp.maximum(s, s * _SLOPE)
    p_self = jnp.exp(jnp.sum(m * att_ref[...], axis=1, keepdims=True))
    num = agg_ref[0] + agg_ref[1] + xl * p_self
    den = den_ref[0] + den_ref[1] + p_self
    a = num / (den + 1e-16) + gb_ref[...]
    a_ref[...] = a

    @pl.when(i == 0)
    def _():
        acc_ref[...] = jnp.zeros_like(acc_ref)

    row = lax.broadcasted_iota(jnp.int32, (8, 128), 0)
    col = lax.broadcasted_iota(jnp.int32, (8, 128), 1)
    upd = jnp.where((row == 0) & (col == 0), jnp.sum(a), 0.0) \
        + jnp.where((row == 0) & (col == 1), jnp.sum(a * a), 0.0)
    acc_ref[...] += upd


def _epi_b_body(a_ref, st_ref, w1_ref, b1_ref, rs_ref, mom_ref):
    i = pl.program_id(0)
    s1 = st_ref[0, 0]
    inv1 = st_ref[0, 1]
    r = jnp.maximum((a_ref[...] - s1) * inv1 * w1_ref[...] + b1_ref[...], 0.0)

    @pl.when(i == 0)
    def _():
        rs_ref[...] = jnp.zeros_like(rs_ref)
        mom_ref[...] = jnp.zeros_like(mom_ref)

    rs_ref[...] += jnp.sum(r, axis=0, keepdims=True)
    mom_ref[...] += jax.lax.dot_general(
        r, r, (((0,), (0,)), ((), ())), preferred_element_type=jnp.float32)


def _epi_c_body(a_ref, h0_ref, st_ref, w1_ref, b1_ref, wlt_ref, blin_ref,
                w2_ref, b2_ref, out_ref):
    s1 = st_ref[0, 0]
    inv1 = st_ref[0, 1]
    s2 = st_ref[0, 2]
    inv2 = st_ref[0, 3]
    r = jnp.maximum((a_ref[...] - s1) * inv1 * w1_ref[...] + b1_ref[...], 0.0)
    t = jnp.dot(r, wlt_ref[...], preferred_element_type=jnp.float32) \
        + blin_ref[...]
    t2 = (t - s2) * inv2 * w2_ref[...] + b2_ref[...]
    h0 = h0_ref[...]
    h1 = jnp.maximum(t2 + h0, 0.0)
    out_ref[...] = jnp.maximum(h0, h1)


def _epilogue(agg_parts, den_parts, x_l, x_r, h0, att, gat_bias,
              norm1_w, norm1_b, W_lin, b_lin, norm2_w, norm2_b):
    full = lambda shape: pl.BlockSpec(shape, lambda i: (0, 0))
    row16 = pl.BlockSpec((B, D), lambda i: (i, 0))

    a, acc = pl.pallas_call(
        _epi_a_body,
        grid=(NB,),
        in_specs=[
            pl.BlockSpec((NC, B, D), lambda i: (0, i, 0)),
            pl.BlockSpec((NC, B, 1), lambda i: (0, i, 0)),
            row16, row16, full((1, D)), full((1, D)),
        ],
        out_specs=[row16, pl.BlockSpec((8, 128), lambda i: (0, 0))],
        out_shape=[jax.ShapeDtypeStruct((N, D), jnp.float32),
                   jax.ShapeDtypeStruct((8, 128), jnp.float32)],
    )(agg_parts, den_parts.reshape(NC, N, 1), x_l, x_r,
      att.reshape(1, D), gat_bias.reshape(1, D))

    cnt = jnp.float32(N * D)
    s1 = acc[0, 0] / cnt
    v1 = acc[0, 1] / cnt - s1 * s1
    inv1 = 1.0 / jnp.sqrt(v1 + EPS)
    stats1 = jnp.zeros((1, 128), jnp.float32)
    stats1 = stats1.at[0, 0].set(s1).at[0, 1].set(inv1)

    r_sum2d, mom = pl.pallas_call(
        _epi_b_body,
        grid=(NB,),
        in_specs=[row16, full((1, 128)), full((1, D)), full((1, D))],
        out_specs=[pl.BlockSpec((1, D), lambda i: (0, 0)),
                   pl.BlockSpec((D, D), lambda i: (0, 0))],
        out_shape=[jax.ShapeDtypeStruct((1, D), jnp.float32),
                   jax.ShapeDtypeStruct((D, D), jnp.float32)],
    )(a, stats1, norm1_w.reshape(1, D), norm1_b.reshape(1, D))

    r_sum = r_sum2d[0]
    wr = W_lin @ r_sum
    sum_t = jnp.sum(wr) + cnt / jnp.float32(D) * jnp.sum(b_lin)
    sum_t2 = jnp.sum((W_lin @ mom) * W_lin) + 2.0 * jnp.dot(b_lin, wr) \
        + cnt / jnp.float32(D) * jnp.sum(b_lin * b_lin)
    s2 = sum_t / cnt
    v2 = sum_t2 / cnt - s2 * s2
    inv2 = 1.0 / jnp.sqrt(v2 + EPS)
    stats = stats1.at[0, 2].set(s2).at[0, 3].set(inv2)

    return pl.pallas_call(
        _epi_c_body,
        grid=(NB,),
        in_specs=[row16, row16, full((1, 128)), full((1, D)), full((1, D)),
                  full((D, D)), full((1, D)), full((1, D)), full((1, D))],
        out_specs=row16,
        out_shape=jax.ShapeDtypeStruct((N, D), jnp.float32),
    )(a, h0, stats, norm1_w.reshape(1, D), norm1_b.reshape(1, D), W_lin.T,
      b_lin.reshape(1, D), norm2_w.reshape(1, D), norm2_b.reshape(1, D))


def kernel(x, edge_index, edge_attr, W_embed, b_embed, ln0_w, ln0_b, W_l, b_l,
           W_r, b_r, W_e, att, gat_bias, norm1_w, norm1_b, W_lin, b_lin,
           norm2_w, norm2_b):
    h0, x_l, x_r = _prologue(x, W_embed, b_embed, ln0_w, ln0_b,
                             W_l, b_l, W_r, b_r)
    src2d = edge_index[0].astype(jnp.int32).reshape(E // SUB, SUB)
    dst2d = edge_index[1].astype(jnp.int32).reshape(E // SUB, SUB)
    dsthi2d = dst2d >> 4
    att1 = att.reshape(D)
    zeros = jnp.zeros((ND, 16), jnp.float32)
    agg_parts, den_parts = _sc_edge_pass(x_l, x_r, src2d, dst2d, dsthi2d,
                                         edge_attr.reshape(E * DE // 128, 128),
                                         W_e.T.reshape(D * DE), att1, zeros)
    return _epilogue(agg_parts, den_parts, x_l, x_r, h0, att1, gat_bias,
                     norm1_w, norm1_b, W_lin, b_lin, norm2_w, norm2_b)


# speedup vs baseline: 1.0195x; 1.0195x over previous
import functools

import jax
import jax.numpy as jnp
from jax import lax
from jax.experimental import pallas as pl
from jax.experimental.pallas import tpu as pltpu
from jax.experimental.pallas import tpu_sc as plsc

N = 100000
E = 3200000
D_IN = 128
D = 16
DE = 4
NC = 2
NS = 16
NW = NC * NS
SUB = 128
KSUB = 2
C = SUB * KSUB
NCHUNK = E // C
ZR = 1000
B = 1000
NB = N // B
EPS = 1e-5
_SLOPE = 0.2


def _prologue_body(x_ref, wet_ref, be_ref, lw_ref, lb_ref, wlt_ref, bl_ref,
                   wrt_ref, br_ref, h0_ref, xl_ref, xr_ref):
    h = jnp.dot(x_ref[...], wet_ref[...], preferred_element_type=jnp.float32)
    h = h + be_ref[...]
    mu = jnp.mean(h, axis=1, keepdims=True)
    var = jnp.mean((h - mu) ** 2, axis=1, keepdims=True)
    h = (h - mu) / jnp.sqrt(var + EPS) * lw_ref[...] + lb_ref[...]
    h0 = jnp.maximum(h, 0.0)
    h0_ref[...] = h0
    xl_ref[...] = jnp.dot(h0, wlt_ref[...],
                          preferred_element_type=jnp.float32) + bl_ref[...]
    xr_ref[...] = jnp.dot(h0, wrt_ref[...],
                          preferred_element_type=jnp.float32) + br_ref[...]


def _prologue(x, W_embed, b_embed, ln0_w, ln0_b, W_l, b_l, W_r, b_r):
    full = lambda shape: pl.BlockSpec(shape, lambda i: (0, 0))
    out = jax.ShapeDtypeStruct((N, D), jnp.float32)
    return pl.pallas_call(
        _prologue_body,
        grid=(NB,),
        in_specs=[
            pl.BlockSpec((B, D_IN), lambda i: (i, 0)),
            full((D_IN, D)), full((1, D)), full((1, D)), full((1, D)),
            full((D, D)), full((1, D)), full((D, D)), full((1, D)),
        ],
        out_specs=[pl.BlockSpec((B, D), lambda i: (i, 0))] * 3,
        out_shape=[out, out, out],
    )(x, W_embed.T, b_embed.reshape(1, D), ln0_w.reshape(1, D),
      ln0_b.reshape(1, D), W_l.T, b_l.reshape(1, D), W_r.T, b_r.reshape(1, D))


ND = N // 16


def _sc_body(xl_hbm, xr_hbm, src_hbm, dst_hbm, dsthi_hbm, ea_hbm, we_hbm,
             att_hbm, zeros_hbm,
             agg_out, den_out,
             agg_sh, den_sh, srcv, dstv, dsthiv, eav, xlv, xrv,
             wev, attv, sem_l, sem_r):
    cid = lax.axis_index("c")
    sid = lax.axis_index("s")
    wid = sid * NC + cid

    pltpu.sync_copy(we_hbm, wev)
    pltpu.sync_copy(att_hbm, attv)
    att_vec = attv[...]
    we_col = [wev[pl.ds(16 * k, 16)] for k in range(DE)]

    iota = lax.iota(jnp.int32, 16)

    @pl.loop(sid, N // ZR, step=NS)
    def _init(i):
        pltpu.sync_copy(zeros_hbm.at[pl.ds(0, ZR)],
                        agg_sh.at[pl.ds(i * ZR, ZR)])

    @pl.when(sid == 0)
    def _init_den():
        pltpu.sync_copy(zeros_hbm, den_sh)

    plsc.subcore_barrier()

    @pl.loop(wid, NCHUNK, step=NW)
    def _chunk(ch):
        pltpu.sync_copy(src_hbm.at[pl.ds(ch * C, C)], srcv)
        pltpu.sync_copy(dst_hbm.at[pl.ds(ch * C, C)], dstv)
        pltpu.sync_copy(dsthi_hbm.at[pl.ds(ch * C, C)], dsthiv)
        pltpu.sync_copy(ea_hbm.at[pl.ds(ch * C * DE, C * DE)], eav)
        for j in range(KSUB):
            pltpu.async_copy(xl_hbm.at[srcv.at[pl.ds(j * SUB, SUB)]],
                             xlv.at[pl.ds(j * SUB, SUB)], sem_l)
            pltpu.async_copy(xr_hbm.at[dstv.at[pl.ds(j * SUB, SUB)]],
                             xrv.at[pl.ds(j * SUB, SUB)], sem_r)
        for j in range(KSUB):
            pltpu.make_async_copy(xl_hbm.at[srcv.at[pl.ds(j * SUB, SUB)]],
                                  xlv.at[pl.ds(j * SUB, SUB)], sem_l).wait()
            pltpu.make_async_copy(xr_hbm.at[dstv.at[pl.ds(j * SUB, SUB)]],
                                  xrv.at[pl.ds(j * SUB, SUB)], sem_r).wait()

        @pl.loop(0, C // 16)
        def _group(g):
            e0 = g * 16
            lanes = dstv[pl.ds(e0, 16)] & 15
            av = jnp.zeros((16,), jnp.float32)
            xls = []
            for sub4 in range(4):
                q = g * 4 + sub4
                ea_vec = eav[pl.ds(q * 16, 16)]
                for q in range(4):
                    r = sub4 * 4 + q
                    xl = xlv[e0 + r]
                    xr = xrv[e0 + r]
                    s = xl + xr
                    for k in range(DE):
                        s = s + ea_vec[q * DE + k] * we_col[k]
                    m = jnp.maximum(s, s * _SLOPE)
                    t = m * att_vec
                    lvs = [t[l] for l in range(16)]
                    while len(lvs) > 1:
                        lvs = [lvs[i] + lvs[i + 1]
                               for i in range(0, len(lvs), 2)]
                    alpha = lvs[0]
                    av = jnp.where(iota == r, alpha, av)
                    xls.append(xl)
            p = jnp.exp(av)
            for r in range(16):
                xlv[e0 + r] = xls[r] * p[r]
                xrv[e0 + r] = jnp.where(iota == lanes[r], p[r], 0.0)

        for j in range(KSUB):
            pltpu.sync_copy(xlv.at[pl.ds(j * SUB, SUB)],
                            agg_sh.at[dstv.at[pl.ds(j * SUB, SUB)]], add=True)
            pltpu.sync_copy(xrv.at[pl.ds(j * SUB, SUB)],
                            den_sh.at[dsthiv.at[pl.ds(j * SUB, SUB)]],
                            add=True)

    plsc.subcore_barrier()

    @pl.loop(sid, N // ZR, step=NS)
    def _dump(i):
        pltpu.sync_copy(agg_sh.at[pl.ds(i * ZR, ZR)],
                        agg_out.at[cid, pl.ds(i * ZR, ZR)])

    @pl.when(sid == 0)
    def _dump_den():
        pltpu.sync_copy(den_sh, den_out.at[cid])


def _sc_edge_pass(x_l, x_r, src2d, dst2d, dsthi2d, edge_attr, W_e, att,
                  zeros):
    mesh = plsc.VectorSubcoreMesh(core_axis_name="c", subcore_axis_name="s")
    fn = pl.kernel(
        _sc_body,
        out_type=[jax.ShapeDtypeStruct((NC, N, D), jnp.float32),
                  jax.ShapeDtypeStruct((NC, ND, 16), jnp.float32)],
        mesh=mesh,
        compiler_params=pltpu.CompilerParams(use_tc_tiling_on_sc=False),
        scratch_types=[
            pltpu.VMEM_SHARED((N, D), jnp.float32),
            pltpu.VMEM_SHARED((ND, 16), jnp.float32),
            pltpu.VMEM((C,), jnp.int32),
            pltpu.VMEM((C,), jnp.int32),
            pltpu.VMEM((C,), jnp.int32),
            pltpu.VMEM((C * DE,), jnp.float32),
            pltpu.VMEM((C, D), jnp.float32),
            pltpu.VMEM((C, D), jnp.float32),
            pltpu.VMEM((D * DE,), jnp.float32),
            pltpu.VMEM((D,), jnp.float32),
            pltpu.SemaphoreType.DMA,
            pltpu.SemaphoreType.DMA,
        ],
    )
    return fn(x_l, x_r, src2d, dst2d, dsthi2d, edge_attr, W_e, att, zeros)


def _epi_a_body(agg_ref, den_ref, xl_ref, xr_ref, att_ref, gb_ref,
                a_ref, acc_ref):
    i = pl.program_id(0)
    xl = xl_ref[...]
    s = xl + xr_ref[...]
    m = jnp.maximum(s, s * _SLOPE)
    p_self = jnp.exp(jnp.sum(m * att_ref[...], axis=1, keepdims=True))
    num = agg_ref[0] + agg_ref[1] + xl * p_self
    den = den_ref[0] + den_ref[1] + p_self
    a = num / (den + 1e-16) + gb_ref[...]
    a_ref[...] = a

    @pl.when(i == 0)
    def _():
        acc_ref[...] = jnp.zeros_like(acc_ref)

    row = lax.broadcasted_iota(jnp.int32, (8, 128), 0)
    col = lax.broadcasted_iota(jnp.int32, (8, 128), 1)
    upd = jnp.where((row == 0) & (col == 0), jnp.sum(a), 0.0) \
        + jnp.where((row == 0) & (col == 1), jnp.sum(a * a), 0.0)
    acc_ref[...] += upd


def _epi_b_body(a_ref, st_ref, w1_ref, b1_ref, rs_ref, mom_ref):
    i = pl.program_id(0)
    s1 = st_ref[0, 0]
    inv1 = st_ref[0, 1]
    r = jnp.maximum((a_ref[...] - s1) * inv1 * w1_ref[...] + b1_ref[...], 0.0)

    @pl.when(i == 0)
    def _():
        rs_ref[...] = jnp.zeros_like(rs_ref)
        mom_ref[...] = jnp.zeros_like(mom_ref)

    rs_ref[...] += jnp.sum(r, axis=0, keepdims=True)
    mom_ref[...] += jax.lax.dot_general(
        r, r, (((0,), (0,)), ((), ())), preferred_element_type=jnp.float32)


def _epi_c_body(a_ref, h0_ref, st_ref, w1_ref, b1_ref, wlt_ref, blin_ref,
                w2_ref, b2_ref, out_ref):
    s1 = st_ref[0, 0]
    inv1 = st_ref[0, 1]
    s2 = st_ref[0, 2]
    inv2 = st_ref[0, 3]
    r = jnp.maximum((a_ref[...] - s1) * inv1 * w1_ref[...] + b1_ref[...], 0.0)
    t = jnp.dot(r, wlt_ref[...], preferred_element_type=jnp.float32) \
        + blin_ref[...]
    t2 = (t - s2) * inv2 * w2_ref[...] + b2_ref[...]
    h0 = h0_ref[...]
    h1 = jnp.maximum(t2 + h0, 0.0)
    out_ref[...] = jnp.maximum(h0, h1)


def _epilogue(agg_parts, den_parts, x_l, x_r, h0, att, gat_bias,
              norm1_w, norm1_b, W_lin, b_lin, norm2_w, norm2_b):
    full = lambda shape: pl.BlockSpec(shape, lambda i: (0, 0))
    row16 = pl.BlockSpec((B, D), lambda i: (i, 0))

    a, acc = pl.pallas_call(
        _epi_a_body,
        grid=(NB,),
        in_specs=[
            pl.BlockSpec((NC, B, D), lambda i: (0, i, 0)),
            pl.BlockSpec((NC, B, 1), lambda i: (0, i, 0)),
            row16, row16, full((1, D)), full((1, D)),
        ],
        out_specs=[row16, pl.BlockSpec((8, 128), lambda i: (0, 0))],
        out_shape=[jax.ShapeDtypeStruct((N, D), jnp.float32),
                   jax.ShapeDtypeStruct((8, 128), jnp.float32)],
    )(agg_parts, den_parts.reshape(NC, N, 1), x_l, x_r,
      att.reshape(1, D), gat_bias.reshape(1, D))

    cnt = jnp.float32(N * D)
    s1 = acc[0, 0] / cnt
    v1 = acc[0, 1] / cnt - s1 * s1
    inv1 = 1.0 / jnp.sqrt(v1 + EPS)
    stats1 = jnp.zeros((1, 128), jnp.float32)
    stats1 = stats1.at[0, 0].set(s1).at[0, 1].set(inv1)

    r_sum2d, mom = pl.pallas_call(
        _epi_b_body,
        grid=(NB,),
        in_specs=[row16, full((1, 128)), full((1, D)), full((1, D))],
        out_specs=[pl.BlockSpec((1, D), lambda i: (0, 0)),
                   pl.BlockSpec((D, D), lambda i: (0, 0))],
        out_shape=[jax.ShapeDtypeStruct((1, D), jnp.float32),
                   jax.ShapeDtypeStruct((D, D), jnp.float32)],
    )(a, stats1, norm1_w.reshape(1, D), norm1_b.reshape(1, D))

    r_sum = r_sum2d[0]
    wr = W_lin @ r_sum
    sum_t = jnp.sum(wr) + cnt / jnp.float32(D) * jnp.sum(b_lin)
    sum_t2 = jnp.sum((W_lin @ mom) * W_lin) + 2.0 * jnp.dot(b_lin, wr) \
        + cnt / jnp.float32(D) * jnp.sum(b_lin * b_lin)
    s2 = sum_t / cnt
    v2 = sum_t2 / cnt - s2 * s2
    inv2 = 1.0 / jnp.sqrt(v2 + EPS)
    stats = stats1.at[0, 2].set(s2).at[0, 3].set(inv2)

    return pl.pallas_call(
        _epi_c_body,
        grid=(NB,),
        in_specs=[row16, row16, full((1, 128)), full((1, D)), full((1, D)),
                  full((D, D)), full((1, D)), full((1, D)), full((1, D))],
        out_specs=row16,
        out_shape=jax.ShapeDtypeStruct((N, D), jnp.float32),
    )(a, h0, stats, norm1_w.reshape(1, D), norm1_b.reshape(1, D), W_lin.T,
      b_lin.reshape(1, D), norm2_w.reshape(1, D), norm2_b.reshape(1, D))


def kernel(x, edge_index, edge_attr, W_embed, b_embed, ln0_w, ln0_b, W_l, b_l,
           W_r, b_r, W_e, att, gat_bias, norm1_w, norm1_b, W_lin, b_lin,
           norm2_w, norm2_b):
    h0, x_l, x_r = _prologue(x, W_embed, b_embed, ln0_w, ln0_b,
                             W_l, b_l, W_r, b_r)
    src1 = edge_index[0].astype(jnp.int32)
    dst1 = edge_index[1].astype(jnp.int32)
    dsthi1 = dst1 >> 4
    att1 = att.reshape(D)
    zeros = jnp.zeros((ND, 16), jnp.float32)
    agg_parts, den_parts = _sc_edge_pass(x_l, x_r, src1, dst1, dsthi1,
                                         edge_attr.reshape(E * DE),
                                         W_e.T.reshape(D * DE), att1, zeros)
    return _epilogue(agg_parts, den_parts, x_l, x_r, h0, att1, gat_bias,
                     norm1_w, norm1_b, W_lin, b_lin, norm2_w, norm2_b)


# speedup vs baseline: 1.1374x; 1.1156x over previous
import functools

import jax
import jax.numpy as jnp
from jax import lax
from jax.experimental import pallas as pl
from jax.experimental.pallas import tpu as pltpu
from jax.experimental.pallas import tpu_sc as plsc

N = 100000
E = 3200000
D_IN = 128
D = 16
DE = 4
NC = 2
NS = 16
NW = NC * NS
SUB = 128
KSUB = 2
C = SUB * KSUB
NCHUNK = E // C
ZR = 1000
B = 1000
NB = N // B
EPS = 1e-5
_SLOPE = 0.2


def _prologue_body(x_ref, wet_ref, be_ref, lw_ref, lb_ref, wlt_ref, bl_ref,
                   wrt_ref, br_ref, h0_ref, xl_ref, xr_ref):
    h = jnp.dot(x_ref[...], wet_ref[...], preferred_element_type=jnp.float32)
    h = h + be_ref[...]
    mu = jnp.mean(h, axis=1, keepdims=True)
    var = jnp.mean((h - mu) ** 2, axis=1, keepdims=True)
    h = (h - mu) / jnp.sqrt(var + EPS) * lw_ref[...] + lb_ref[...]
    h0 = jnp.maximum(h, 0.0)
    h0_ref[...] = h0
    xl_ref[...] = jnp.dot(h0, wlt_ref[...],
                          preferred_element_type=jnp.float32) + bl_ref[...]
    xr_ref[...] = jnp.dot(h0, wrt_ref[...],
                          preferred_element_type=jnp.float32) + br_ref[...]


def _prologue(x, W_embed, b_embed, ln0_w, ln0_b, W_l, b_l, W_r, b_r):
    full = lambda shape: pl.BlockSpec(shape, lambda i: (0, 0))
    out = jax.ShapeDtypeStruct((N, D), jnp.float32)
    return pl.pallas_call(
        _prologue_body,
        grid=(NB,),
        in_specs=[
            pl.BlockSpec((B, D_IN), lambda i: (i, 0)),
            full((D_IN, D)), full((1, D)), full((1, D)), full((1, D)),
            full((D, D)), full((1, D)), full((D, D)), full((1, D)),
        ],
        out_specs=[pl.BlockSpec((B, D), lambda i: (i, 0))] * 3,
        out_shape=[out, out, out],
    )(x, W_embed.T, b_embed.reshape(1, D), ln0_w.reshape(1, D),
      ln0_b.reshape(1, D), W_l.T, b_l.reshape(1, D), W_r.T, b_r.reshape(1, D))


ND = N // 16


def _sc_body(xl_hbm, xr_hbm, src_hbm, dst_hbm, ea_hbm, we_hbm,
             att_hbm, zeros_hbm,
             agg_out, den_out,
             agg_sh, den_sh,
             srcv0, srcv1, dstv0, dstv1, dsthiv0, dsthiv1, eav0, eav1,
             xlv0, xlv1, xrv0, xrv1, wev, attv,
             sem_lin0, sem_lin1, sem_gl0, sem_gl1, sem_gr0, sem_gr1,
             sem_o0, sem_o1):
    cid = lax.axis_index("c")
    sid = lax.axis_index("s")
    wid = sid * NC + cid

    srcv = [srcv0, srcv1]
    dstv = [dstv0, dstv1]
    dsthiv = [dsthiv0, dsthiv1]
    eav = [eav0, eav1]
    xlv = [xlv0, xlv1]
    xrv = [xrv0, xrv1]
    sem_lin = [sem_lin0, sem_lin1]
    sem_gl = [sem_gl0, sem_gl1]
    sem_gr = [sem_gr0, sem_gr1]
    sem_o = [sem_o0, sem_o1]

    pltpu.sync_copy(we_hbm, wev)
    pltpu.sync_copy(att_hbm, attv)
    att_vec = attv[...]
    we_col = [wev[pl.ds(16 * k, 16)] for k in range(DE)]

    iota = lax.iota(jnp.int32, 16)

    @pl.loop(sid, N // ZR, step=NS)
    def _init(i):
        pltpu.sync_copy(zeros_hbm.at[pl.ds(0, ZR)],
                        agg_sh.at[pl.ds(i * ZR, ZR)])

    @pl.when(sid == 0)
    def _init_den():
        pltpu.sync_copy(zeros_hbm, den_sh)

    plsc.subcore_barrier()

    def lin_copies(ch, b):
        return [
            pltpu.make_async_copy(src_hbm.at[pl.ds(ch * C, C)], srcv[b],
                                  sem_lin[b]),
            pltpu.make_async_copy(dst_hbm.at[pl.ds(ch * C, C)], dstv[b],
                                  sem_lin[b]),
            pltpu.make_async_copy(ea_hbm.at[pl.ds(ch * C * DE, C * DE)],
                                  eav[b], sem_lin[b]),
        ]

    def gather_copies(b):
        out = []
        for j in range(KSUB):
            sl = pl.ds(j * SUB, SUB)
            out.append(pltpu.make_async_copy(
                xl_hbm.at[srcv[b].at[sl]], xlv[b].at[sl], sem_gl[b]))
            out.append(pltpu.make_async_copy(
                xr_hbm.at[dstv[b].at[sl]], xrv[b].at[sl], sem_gr[b]))
        return out

    def scatter_copies(b):
        out = []
        for j in range(KSUB):
            sl = pl.ds(j * SUB, SUB)
            out.append(pltpu.make_async_copy(
                xlv[b].at[sl], agg_sh.at[dstv[b].at[sl]], sem_o[b]))
            out.append(pltpu.make_async_copy(
                xrv[b].at[sl], den_sh.at[dsthiv[b].at[sl]], sem_o[b]))
        return out

    def compute(b):
        @pl.loop(0, C // 16)
        def _group(g):
            e0 = g * 16
            dv = dstv[b][pl.ds(e0, 16)]
            lanes = dv & 15
            dsthiv[b][pl.ds(e0, 16)] = lax.shift_right_logical(dv, 4)
            av = jnp.zeros((16,), jnp.float32)
            xls = []
            for sub4 in range(4):
                ea_vec = eav[b][pl.ds((g * 4 + sub4) * 16, 16)]
                for q in range(4):
                    r = sub4 * 4 + q
                    xl = xlv[b][e0 + r]
                    xr = xrv[b][e0 + r]
                    s = xl + xr
                    for k in range(DE):
                        s = s + ea_vec[q * DE + k] * we_col[k]
                    m = jnp.maximum(s, s * _SLOPE)
                    t = m * att_vec
                    lvs = [t[l] for l in range(16)]
                    while len(lvs) > 1:
                        lvs = [lvs[i] + lvs[i + 1]
                               for i in range(0, len(lvs), 2)]
                    av = jnp.where(iota == r, lvs[0], av)
                    xls.append(xl)
            p = jnp.exp(av)
            for r in range(16):
                xlv[b][e0 + r] = xls[r] * p[r]
                xrv[b][e0 + r] = jnp.where(iota == lanes[r], p[r], 0.0)

    def step(ch, b, chn):
        for cpy in lin_copies(ch, b):
            cpy.wait()
        for cpy in gather_copies(b):
            cpy.start()
        @pl.when(chn < NCHUNK)
        def _pref():
            @pl.when(ch > wid)
            def _drain():
                for cpy in scatter_copies(1 - b):
                    cpy.wait()
            for cpy in lin_copies(chn, 1 - b):
                cpy.start()
        for cpy in gather_copies(b):
            cpy.wait()
        compute(b)
        for cpy in scatter_copies(b):
            cpy.start(add=True)

    for cpy in lin_copies(wid, 0):
        cpy.start()

    @pl.loop(wid, NCHUNK, step=2 * NW)
    def _pair(ch):
        step(ch, 0, ch + NW)

        @pl.when(ch + NW < NCHUNK)
        def _second():
            step(ch + NW, 1, ch + 2 * NW)

    for b in range(2):
        for cpy in scatter_copies(b):
            cpy.wait()

    plsc.subcore_barrier()

    @pl.loop(sid, N // ZR, step=NS)
    def _dump(i):
        pltpu.sync_copy(agg_sh.at[pl.ds(i * ZR, ZR)],
                        agg_out.at[cid, pl.ds(i * ZR, ZR)])

    @pl.when(sid == 0)
    def _dump_den():
        pltpu.sync_copy(den_sh, den_out.at[cid])


def _sc_edge_pass(x_l, x_r, src1, dst1, edge_attr, W_e, att, zeros):
    mesh = plsc.VectorSubcoreMesh(core_axis_name="c", subcore_axis_name="s")
    buf = lambda shape: pltpu.VMEM(shape, jnp.float32)
    ibuf = lambda shape: pltpu.VMEM(shape, jnp.int32)
    fn = pl.kernel(
        _sc_body,
        out_type=[jax.ShapeDtypeStruct((NC, N, D), jnp.float32),
                  jax.ShapeDtypeStruct((NC, ND, 16), jnp.float32)],
        mesh=mesh,
        compiler_params=pltpu.CompilerParams(use_tc_tiling_on_sc=False),
        scratch_types=[
            pltpu.VMEM_SHARED((N, D), jnp.float32),
            pltpu.VMEM_SHARED((ND, 16), jnp.float32),
            ibuf((C,)), ibuf((C,)), ibuf((C,)), ibuf((C,)),
            ibuf((C,)), ibuf((C,)),
            buf((C * DE,)), buf((C * DE,)),
            buf((C, D)), buf((C, D)), buf((C, D)), buf((C, D)),
            buf((D * DE,)), buf((D,)),
        ] + [pltpu.SemaphoreType.DMA] * 8,
    )
    return fn(x_l, x_r, src1, dst1, edge_attr, W_e, att, zeros)


def _epi_a_body(agg_ref, den_ref, xl_ref, xr_ref, att_ref, gb_ref,
                a_ref, acc_ref):
    i = pl.program_id(0)
    xl = xl_ref[...]
    s = xl + xr_ref[...]
    m = jnp.maximum(s, s * _SLOPE)
    p_self = jnp.exp(jnp.sum(m * att_ref[...], axis=1, keepdims=True))
    num = agg_ref[0] + agg_ref[1] + xl * p_self
    den = den_ref[0] + den_ref[1] + p_self
    a = num / (den + 1e-16) + gb_ref[...]
    a_ref[...] = a

    @pl.when(i == 0)
    def _():
        acc_ref[...] = jnp.zeros_like(acc_ref)

    row = lax.broadcasted_iota(jnp.int32, (8, 128), 0)
    col = lax.broadcasted_iota(jnp.int32, (8, 128), 1)
    upd = jnp.where((row == 0) & (col == 0), jnp.sum(a), 0.0) \
        + jnp.where((row == 0) & (col == 1), jnp.sum(a * a), 0.0)
    acc_ref[...] += upd


def _epi_b_body(a_ref, st_ref, w1_ref, b1_ref, rs_ref, mom_ref):
    i = pl.program_id(0)
    s1 = st_ref[0, 0]
    inv1 = st_ref[0, 1]
    r = jnp.maximum((a_ref[...] - s1) * inv1 * w1_ref[...] + b1_ref[...], 0.0)

    @pl.when(i == 0)
    def _():
        rs_ref[...] = jnp.zeros_like(rs_ref)
        mom_ref[...] = jnp.zeros_like(mom_ref)

    rs_ref[...] += jnp.sum(r, axis=0, keepdims=True)
    mom_ref[...] += jax.lax.dot_general(
        r, r, (((0,), (0,)), ((), ())), preferred_element_type=jnp.float32)


def _epi_c_body(a_ref, h0_ref, st_ref, w1_ref, b1_ref, wlt_ref, blin_ref,
                w2_ref, b2_ref, out_ref):
    s1 = st_ref[0, 0]
    inv1 = st_ref[0, 1]
    s2 = st_ref[0, 2]
    inv2 = st_ref[0, 3]
    r = jnp.maximum((a_ref[...] - s1) * inv1 * w1_ref[...] + b1_ref[...], 0.0)
    t = jnp.dot(r, wlt_ref[...], preferred_element_type=jnp.float32) \
        + blin_ref[...]
    t2 = (t - s2) * inv2 * w2_ref[...] + b2_ref[...]
    h0 = h0_ref[...]
    h1 = jnp.maximum(t2 + h0, 0.0)
    out_ref[...] = jnp.maximum(h0, h1)


def _epilogue(agg_parts, den_parts, x_l, x_r, h0, att, gat_bias,
              norm1_w, norm1_b, W_lin, b_lin, norm2_w, norm2_b):
    full = lambda shape: pl.BlockSpec(shape, lambda i: (0, 0))
    row16 = pl.BlockSpec((B, D), lambda i: (i, 0))

    a, acc = pl.pallas_call(
        _epi_a_body,
        grid=(NB,),
        in_specs=[
            pl.BlockSpec((NC, B, D), lambda i: (0, i, 0)),
            pl.BlockSpec((NC, B, 1), lambda i: (0, i, 0)),
            row16, row16, full((1, D)), full((1, D)),
        ],
        out_specs=[row16, pl.BlockSpec((8, 128), lambda i: (0, 0))],
        out_shape=[jax.ShapeDtypeStruct((N, D), jnp.float32),
                   jax.ShapeDtypeStruct((8, 128), jnp.float32)],
    )(agg_parts, den_parts.reshape(NC, N, 1), x_l, x_r,
      att.reshape(1, D), gat_bias.reshape(1, D))

    cnt = jnp.float32(N * D)
    s1 = acc[0, 0] / cnt
    v1 = acc[0, 1] / cnt - s1 * s1
    inv1 = 1.0 / jnp.sqrt(v1 + EPS)
    stats1 = jnp.zeros((1, 128), jnp.float32)
    stats1 = stats1.at[0, 0].set(s1).at[0, 1].set(inv1)

    r_sum2d, mom = pl.pallas_call(
        _epi_b_body,
        grid=(NB,),
        in_specs=[row16, full((1, 128)), full((1, D)), full((1, D))],
        out_specs=[pl.BlockSpec((1, D), lambda i: (0, 0)),
                   pl.BlockSpec((D, D), lambda i: (0, 0))],
        out_shape=[jax.ShapeDtypeStruct((1, D), jnp.float32),
                   jax.ShapeDtypeStruct((D, D), jnp.float32)],
    )(a, stats1, norm1_w.reshape(1, D), norm1_b.reshape(1, D))

    r_sum = r_sum2d[0]
    wr = W_lin @ r_sum
    sum_t = jnp.sum(wr) + cnt / jnp.float32(D) * jnp.sum(b_lin)
    sum_t2 = jnp.sum((W_lin @ mom) * W_lin) + 2.0 * jnp.dot(b_lin, wr) \
        + cnt / jnp.float32(D) * jnp.sum(b_lin * b_lin)
    s2 = sum_t / cnt
    v2 = sum_t2 / cnt - s2 * s2
    inv2 = 1.0 / jnp.sqrt(v2 + EPS)
    stats = stats1.at[0, 2].set(s2).at[0, 3].set(inv2)

    return pl.pallas_call(
        _epi_c_body,
        grid=(NB,),
        in_specs=[row16, row16, full((1, 128)), full((1, D)), full((1, D)),
                  full((D, D)), full((1, D)), full((1, D)), full((1, D))],
        out_specs=row16,
        out_shape=jax.ShapeDtypeStruct((N, D), jnp.float32),
    )(a, h0, stats, norm1_w.reshape(1, D), norm1_b.reshape(1, D), W_lin.T,
      b_lin.reshape(1, D), norm2_w.reshape(1, D), norm2_b.reshape(1, D))


def kernel(x, edge_index, edge_attr, W_embed, b_embed, ln0_w, ln0_b, W_l, b_l,
           W_r, b_r, W_e, att, gat_bias, norm1_w, norm1_b, W_lin, b_lin,
           norm2_w, norm2_b):
    h0, x_l, x_r = _prologue(x, W_embed, b_embed, ln0_w, ln0_b,
                             W_l, b_l, W_r, b_r)
    src1 = edge_index[0].astype(jnp.int32)
    dst1 = edge_index[1].astype(jnp.int32)
    att1 = att.reshape(D)
    zeros = jnp.zeros((ND, 16), jnp.float32)
    agg_parts, den_parts = _sc_edge_pass(x_l, x_r, src1, dst1,
                                         edge_attr.reshape(E * DE),
                                         W_e.T.reshape(D * DE), att1, zeros)
    return _epilogue(agg_parts, den_parts, x_l, x_r, h0, att1, gat_bias,
                     norm1_w, norm1_b, W_lin, b_lin, norm2_w, norm2_b)
